# baseline (speedup 1.0000x reference)
import jax, jax.numpy as jnp
from jax import lax
from jax.experimental import pallas as pl
from jax.experimental.pallas import tpu as pltpu
from jax.experimental.pallas import tpu_sc as plsc

B=16384; DIM=64; NC=2; NS=16; NW=32; BPW=512

def _body(e_t, out, cv, ov, sem):
    wid = lax.axis_index("s") * NC + lax.axis_index("c")
    base = wid * BPW
    cp = pltpu.async_copy(e_t.at[pl.ds(0, 8), pl.ds(wid * 128, 128)], cv, sem)
    cp.wait()
    acc = cv[0, pl.ds(0, 16)]
    ov[pl.ds(0, 16)] = acc
    pltpu.sync_copy(ov, out.at[pl.ds(base, 16)])

def kernel(pos_s, pos_r, pos_o, pos_t, neg_s, neg_r, neg_o, neg_t,
           e_weight, r_weight, t_weight):
    mesh = plsc.VectorSubcoreMesh(core_axis_name="c", subcore_axis_name="s",
                                  num_cores=NC, num_subcores=NS)
    run = pl.kernel(_body,
        out_type=jax.ShapeDtypeStruct((B,), jnp.float32),
        mesh=mesh,
        scratch_types=[pltpu.VMEM((8, 128), jnp.float32),
                       pltpu.VMEM((16,), jnp.float32),
                       pltpu.SemaphoreType.DMA],
        compiler_params=pltpu.CompilerParams(
            needs_layout_passes=False, use_tc_tiling_on_sc=False))
    e_t = e_weight.T
    return (run(e_t), run(e_t))


# speedup vs baseline: 1.5785x; 1.5785x over previous
import jax
import jax.numpy as jnp
from jax import lax
from jax.experimental import pallas as pl
from jax.experimental.pallas import tpu as pltpu
from jax.experimental.pallas import tpu_sc as plsc

B = 16384
DIM = 64
E_CNT = 1000000
PDIM = 128
NC = 2
NS = 16
NW = NC * NS
NBLK = 7936
BPWK = NBLK // NW
FULL_BLK = E_CNT // 128
TAIL_BASE = FULL_BLK * 128
ROWS = 1000072
DUMP = 1000064
BPW = B // NW
CHUNK = 64
NTASK = 2 * BPW // CHUNK


def _extract_body(e_t, hits, tail, rows,
                  hm_v, tv, tail_v, asm_v, idx_v, sem):
    wid = lax.axis_index("s") * NC + lax.axis_index("c")
    lane = lax.iota(jnp.int32, 16)
    first = lane == 0

    pltpu.sync_copy(hits.at[pl.ds(wid * (BPWK * 128), BPWK * 128)], hm_v)
    pltpu.sync_copy(tail, tail_v)

    def process(b, b_local, src, n_groups):
        for j in range(8):
            idx_v[j, :] = jnp.full((16,), DUMP, jnp.int32)

        def grp(lg, slot):
            hv = hm_v[pl.ds(b_local * 128 + lg * 16, 16)]
            cnt = jnp.sum(hv)
            pre = plsc.cumsum(hv) - hv

            @pl.when(cnt > 0)
            def _():
                for k in range(16):
                    hit = hv[k] > 0
                    s = slot + pre[k]
                    colv = jnp.full((16,), lg * 16 + k, jnp.int32)
                    for q in range(DIM // 16):
                        vals = plsc.load_gather(src, [q * 16 + lane, colv])
                        asm_v[s, pl.ds(q * 16, 16)] = vals
                    e_lane = b * 128 + lg * 16 + k
                    plsc.store_scatter(
                        idx_v,
                        [jnp.full((16,), s // 16, jnp.int32),
                         jnp.full((16,), s % 16, jnp.int32)],
                        jnp.full((16,), e_lane, jnp.int32),
                        mask=first & hit)

            return slot + cnt

        slot = lax.fori_loop(0, n_groups, grp, 0)
        for j in range(8):
            @pl.when(slot > j * 16)
            def _():
                pltpu.async_copy(asm_v.at[pl.ds(j * 16, 16)],
                                 rows.at[idx_v.at[j]], sem).wait()

    def block_body(b_local, carry):
        b = wid * BPWK + b_local

        @pl.when(b < FULL_BLK)
        def _():
            off = pl.multiple_of(b * 128, 128)
            pltpu.sync_copy(e_t.at[:, pl.ds(off, 128)], tv)
            process(b, b_local, tv, 8)

        @pl.when(b == FULL_BLK)
        def _():
            process(b, b_local, tail_v, 4)

        return carry

    lax.fori_loop(0, BPWK, block_body, 0)


def _score_body(s_h, o_h, r_h, rp_h, t_h, tp_h,
                rows, r_p, t_p, pos_out, neg_out,
                s_iv, o_iv, r_iv, rp_v, t_iv, tp_v,
                s_v0, r_v0, t_v0, o_v0, s_v1, r_v1, t_v1, o_v1,
                out_v, sem0, sem1):
    wid = lax.axis_index("s") * NC + lax.axis_index("c")
    base = wid * BPW
    lane = lax.iota(jnp.int32, 16)

    for hbm, vmem in ((s_h, s_iv), (o_h, o_iv), (r_h, r_iv),
                      (rp_h, rp_v), (t_h, t_iv), (tp_h, tp_v)):
        pltpu.sync_copy(hbm.at[pl.ds(base, BPW)], vmem.at[pl.ds(0, BPW)])
        pltpu.sync_copy(hbm.at[pl.ds(B + base, BPW)],
                        vmem.at[pl.ds(BPW, BPW)])

    bufs = ((s_v0, r_v0, t_v0, o_v0), (s_v1, r_v1, t_v1, o_v1))
    sems = (sem0, sem1)

    def fire(k):
        sb, rb, tb, ob = bufs[k % 2]
        sem = sems[k % 2]
        cb = k * CHUNK
        return (pltpu.async_copy(rows.at[s_iv.at[pl.ds(cb, CHUNK)]], sb, sem),
                pltpu.async_copy(r_p.at[r_iv.at[pl.ds(cb, CHUNK)]], rb, sem),
                pltpu.async_copy(t_p.at[t_iv.at[pl.ds(cb, CHUNK)]], tb, sem),
                pltpu.async_copy(rows.at[o_iv.at[pl.ds(cb, CHUNK)]], ob, sem))

    def score(k):
        sb, rb, tb, ob = bufs[k % 2]
        tb_off = k * CHUNK

        def group(g, carry):
            gb = tb_off + g * 16
            ir = g * 16 + lane
            rp16 = rp_v[pl.ds(gb, 16)] * DIM
            tp16 = tp_v[pl.ds(gb, 16)] * DIM

            def dim_body(d, acc):
                dv = jnp.full((16,), d, jnp.int32)
                sv = plsc.load_gather(sb, [ir, dv])
                ov = plsc.load_gather(ob, [ir, dv])
                rv = plsc.load_gather(rb, [ir, rp16 + d])
                tv_ = plsc.load_gather(tb, [ir, tp16 + d])
                return acc + jnp.abs(sv + rv + tv_ - ov)

            acc = lax.fori_loop(0, DIM, dim_body,
                                jnp.zeros((16,), jnp.float32))
            out_v[pl.ds(gb, 16)] = acc
            return carry

        lax.fori_loop(0, CHUNK // 16, group, 0)

    pending = fire(0)
    for k in range(NTASK):
        for cp in pending:
            cp.wait()
        if k + 1 < NTASK:
            nxt = fire(k + 1)
        score(k)
        if k + 1 < NTASK:
            pending = nxt

    pltpu.sync_copy(out_v.at[pl.ds(0, BPW)], pos_out.at[pl.ds(base, BPW)])
    pltpu.sync_copy(out_v.at[pl.ds(BPW, BPW)], neg_out.at[pl.ds(base, BPW)])


def kernel(pos_s, pos_r, pos_o, pos_t, neg_s, neg_r, neg_o, neg_t,
           e_weight, r_weight, t_weight):
    mesh = plsc.VectorSubcoreMesh(
        core_axis_name="c", subcore_axis_name="s",
        num_cores=NC, num_subcores=NS)
    f32 = jnp.float32
    i32 = jnp.int32

    extract = pl.kernel(
        _extract_body,
        out_type=jax.ShapeDtypeStruct((ROWS, PDIM), f32),
        mesh=mesh,
        scratch_types=[
            pltpu.VMEM((BPWK * 128,), i32),
            pltpu.VMEM((DIM, 128), f32),
            pltpu.VMEM((DIM, DIM), f32),
            pltpu.VMEM((136, PDIM), f32),
            pltpu.VMEM((8, 16), i32),
            pltpu.SemaphoreType.DMA,
        ],
        compiler_params=pltpu.CompilerParams(needs_layout_passes=False),
    )

    score = pl.kernel(
        _score_body,
        out_type=(jax.ShapeDtypeStruct((B,), f32),
                  jax.ShapeDtypeStruct((B,), f32)),
        mesh=mesh,
        scratch_types=(
            [pltpu.VMEM((2 * BPW,), i32)] * 6
            + [pltpu.VMEM((CHUNK, PDIM), f32)] * 8
            + [pltpu.VMEM((2 * BPW,), f32)]
            + [pltpu.SemaphoreType.DMA] * 2
        ),
        compiler_params=pltpu.CompilerParams(needs_layout_passes=False),
    )

    i = lambda a: a.astype(i32)
    s_all = jnp.concatenate([i(pos_s), i(neg_s)])
    o_all = jnp.concatenate([i(pos_o), i(neg_o)])
    r_all = jnp.concatenate([i(pos_r), i(neg_r)])
    t_all = jnp.concatenate([i(pos_t), i(pos_t)])

    gl = jnp.concatenate([s_all, o_all])
    hits = jnp.zeros((NBLK * 128,), i32).at[gl].set(1)
    e_t = e_weight.T
    tail_t = e_weight[TAIL_BASE:].T
    r_p = r_weight.reshape(r_weight.shape[0] // 2, PDIM)
    t_p = t_weight.reshape(t_weight.shape[0] // 2, PDIM)

    rows = extract(e_t, hits, tail_t)
    return score(s_all, o_all, r_all // 2, r_all & 1, t_all // 2, t_all & 1,
                 rows, r_p, t_p)
